# baseline (speedup 1.0000x reference)
import functools

import jax
import jax.numpy as jnp
from jax import lax
from jax.experimental import pallas as pl
from jax.experimental.pallas import tpu as pltpu
from jax.experimental.pallas import tpu_sc as plsc

V = 10000
E = 160000
DIN = 128
DOUT = 32
DE = 16
DH = 128
STEPS = 3

VP = 10240
NC = 2
NS = 16
NW = NC * NS
CHUNK = 1000
EA = 32000
EPWA = EA // NW
NCHA = EPWA // CHUNK
EPWB = (E - EA) // NW
NCHB = EPWB // CHUNK

TE = 4000
TEM = 4000
BV = 2048

@functools.lru_cache(maxsize=None)
def _sc_mesh():
    return plsc.VectorSubcoreMesh(
        core_axis_name="c", subcore_axis_name="s", num_cores=NC, num_subcores=NS
    )


def _pack4(y, q):
    return jnp.concatenate([y[k * q:(k + 1) * q, :] for k in range(4)], axis=1)


def _unpack4(yp):
    return jnp.concatenate(
        [yp[:, k * DOUT:(k + 1) * DOUT] for k in range(4)], axis=0)


def _proj_body(nf_ref, w_ref, b_ref, o_ref):
    y = jax.nn.relu(
        jnp.dot(nf_ref[...], w_ref[...], preferred_element_type=jnp.float32)
        + b_ref[...]
    )
    o_ref[...] = _pack4(y, BV // 4)


def _proj(nf_perm, W_proj, b_proj):
    return pl.pallas_call(
        _proj_body,
        grid=(VP // BV,),
        in_specs=[
            pl.BlockSpec((BV, DIN), lambda i: (i, 0)),
            pl.BlockSpec((DIN, DOUT), lambda i: (0, 0)),
            pl.BlockSpec((1, DOUT), lambda i: (0, 0)),
        ],
        out_specs=pl.BlockSpec((BV // 4, 128), lambda i: (i, 0)),
        out_shape=jax.ShapeDtypeStruct((VP // 4, 128), jnp.float32),
    )(nf_perm, W_proj, b_proj.reshape(1, DOUT))


def _make_gather(off, epw, nch):
    def body(x_hbm, src_hbm, out_hbm, idx_v, rows0, rows1, gs0, gs1, ws0, ws1):
        c = lax.axis_index("c")
        s = lax.axis_index("s")
        base = (c * NS + s) * epw
        pltpu.sync_copy(src_hbm.at[pl.ds(off + base, epw)], idx_v)
        bufs = (rows0, rows1)
        gsems = (gs0, gs1)
        wsems = (ws0, ws1)
        gcp = [None] * nch
        wcp = [None] * nch
        gcp[0] = pltpu.async_copy(
            x_hbm.at[idx_v.at[pl.ds(0, CHUNK)]], bufs[0], gsems[0])
        for j in range(nch):
            b = j % 2
            if j + 1 < nch:
                if j >= 1:
                    wcp[j - 1].wait()
                gcp[j + 1] = pltpu.async_copy(
                    x_hbm.at[idx_v.at[pl.ds((j + 1) * CHUNK, CHUNK)]],
                    bufs[1 - b], gsems[1 - b])
            gcp[j].wait()
            wcp[j] = pltpu.async_copy(
                bufs[b], out_hbm.at[pl.ds(base + j * CHUNK, CHUNK)], wsems[b])
        if nch >= 2:
            wcp[nch - 2].wait()
        wcp[nch - 1].wait()

    return pl.kernel(
        body,
        out_type=jax.ShapeDtypeStruct((epw * NW, DOUT), jnp.float32),
        mesh=_sc_mesh(),
        scratch_types=[
            pltpu.VMEM((epw,), jnp.int32),
            pltpu.VMEM((CHUNK, DOUT), jnp.float32),
            pltpu.VMEM((CHUNK, DOUT), jnp.float32),
            pltpu.SemaphoreType.DMA,
            pltpu.SemaphoreType.DMA,
            pltpu.SemaphoreType.DMA,
            pltpu.SemaphoreType.DMA,
        ],
        compiler_params=pltpu.CompilerParams(use_tc_tiling_on_sc=False),
    )


@functools.lru_cache(maxsize=None)
def _gather_kernels():
    return (_make_gather(0, EPWA, NCHA), _make_gather(EA, EPWB, NCHB))


def _make_scatter(epw, nch):
    def body(msg_hbm, dst_hbm, zeros_hbm, out_hbm, idx2_v, rows0, rows1,
             acc_sh, ls0, ls1, ss0, ss1):
        c = lax.axis_index("c")
        s = lax.axis_index("s")
        rpt = VP // NS
        base = (c * NS + s) * epw
        zcp = pltpu.async_copy(
            zeros_hbm.at[pl.ds(s * rpt, rpt)], acc_sh.at[pl.ds(s * rpt, rpt)],
            ss0)
        pltpu.sync_copy(dst_hbm.at[c * NS + s], idx2_v)
        zcp.wait()
        plsc.subcore_barrier()
        bufs = (rows0, rows1)
        lsems = (ls0, ls1)
        ssems = (ss0, ss1)
        lcp = [None] * nch
        scp = [None] * nch
        lcp[0] = pltpu.async_copy(
            msg_hbm.at[pl.ds(base, CHUNK)], bufs[0], lsems[0])
        for j in range(nch):
            b = j % 2
            if j + 1 < nch:
                if j >= 1:
                    scp[j - 1].wait()
                lcp[j + 1] = pltpu.async_copy(
                    msg_hbm.at[pl.ds(base + (j + 1) * CHUNK, CHUNK)],
                    bufs[1 - b], lsems[1 - b])
            lcp[j].wait()
            scp[j] = pltpu.async_copy(
                bufs[b], acc_sh.at[idx2_v.at[j]], ssems[b], add=True)
        if nch >= 2:
            scp[nch - 2].wait()
        scp[nch - 1].wait()
        plsc.subcore_barrier()
        pltpu.sync_copy(
            acc_sh.at[pl.ds(s * rpt, rpt)], out_hbm.at[c, pl.ds(s * rpt, rpt)]
        )

    return pl.kernel(
        body,
        out_type=jax.ShapeDtypeStruct((NC, VP, DOUT), jnp.float32),
        mesh=_sc_mesh(),
        scratch_types=[
            pltpu.VMEM((nch, CHUNK), jnp.int32),
            pltpu.VMEM((CHUNK, DOUT), jnp.float32),
            pltpu.VMEM((CHUNK, DOUT), jnp.float32),
            pltpu.VMEM_SHARED((VP, DOUT), jnp.float32),
            pltpu.SemaphoreType.DMA,
            pltpu.SemaphoreType.DMA,
            pltpu.SemaphoreType.DMA,
            pltpu.SemaphoreType.DMA,
        ],
        compiler_params=pltpu.CompilerParams(use_tc_tiling_on_sc=False),
    )


@functools.lru_cache(maxsize=None)
def _scatter_kernels():
    return (_make_scatter(EPWA, NCHA), _make_scatter(EPWB, NCHB))


def _wbuild_body(ea_ref, we1_ref, be1_ref, we2p_ref, o_ref):
    he = jax.nn.relu(
        jnp.dot(ea_ref[...].astype(jnp.bfloat16), we1_ref[...],
                preferred_element_type=jnp.float32)
        + be1_ref[...]
    ).astype(jnp.bfloat16)
    o_ref[...] = jnp.dot(
        he, we2p_ref[...], preferred_element_type=jnp.float32
    ).astype(jnp.bfloat16)


def _wbuild(edge_attr_f32, W_e1, b_e1, we2p):
    return pl.pallas_call(
        _wbuild_body,
        grid=(E // TE,),
        in_specs=[
            pl.BlockSpec((TE, DE), lambda i: (i, 0)),
            pl.BlockSpec((DE, DH), lambda i: (0, 0)),
            pl.BlockSpec((1, DH), lambda i: (0, 0)),
            pl.BlockSpec((DH, DOUT * DOUT), lambda i: (0, 0)),
        ],
        out_specs=pl.BlockSpec((TE, DOUT * DOUT), lambda i: (i, 0)),
        out_shape=jax.ShapeDtypeStruct((E, DOUT * DOUT), jnp.bfloat16),
    )(edge_attr_f32, W_e1, b_e1.reshape(1, DH), we2p)


def _msg_body(wep_ref, xjp_ref, rep4_ref, b2m_ref, red_ref, o_ref):
    q = TEM // 4
    we = wep_ref[...].astype(jnp.bfloat16)
    xjp = xjp_ref[...].astype(jnp.bfloat16)
    xj = jnp.concatenate(
        [xjp[:, k * DOUT:(k + 1) * DOUT] for k in range(4)], axis=0)
    xt = jnp.dot(xj, rep4_ref[...],
                 preferred_element_type=jnp.float32).astype(jnp.bfloat16)
    xr = jnp.concatenate([xt] * (DOUT * DOUT // 128), axis=1)
    p = we * xr
    msg = (
        jnp.dot(p, red_ref[...], preferred_element_type=jnp.float32)
        + jnp.dot(xj, b2m_ref[...], preferred_element_type=jnp.float32)
    )
    o_ref[...] = jnp.concatenate(
        [msg[k * q:(k + 1) * q, :] for k in range(4)], axis=1)


def _msg(wep, xjp, rep4, b2m, red, off_b, nb):
    return pl.pallas_call(
        _msg_body,
        grid=(nb,),
        in_specs=[
            pl.BlockSpec((TEM, DOUT * DOUT), lambda i: (i + off_b, 0)),
            pl.BlockSpec((TEM // 4, 128), lambda i: (i, 0)),
            pl.BlockSpec((DOUT, 128), lambda i: (0, 0)),
            pl.BlockSpec((DOUT, DOUT), lambda i: (0, 0)),
            pl.BlockSpec((DOUT * DOUT, DOUT), lambda i: (0, 0)),
        ],
        out_specs=pl.BlockSpec((TEM // 4, 128), lambda i: (i, 0)),
        out_shape=jax.ShapeDtypeStruct((nb * TEM // 4, 128), jnp.float32),
    )(wep, xjp, rep4, b2m, red)


def _gru_body(a_ref, b_agg_ref, x_ref, wr_ref, bc_ref, wir_ref, wiz_ref,
              win_ref, whr_ref, whz_ref, whn_ref, bi_ref, bh_ref, o_ref):
    x = _unpack4(x_ref[...])
    conv = (
        _unpack4(a_ref[0]) + _unpack4(a_ref[1])
        + _unpack4(b_agg_ref[0]) + _unpack4(b_agg_ref[1])
        + jnp.dot(x, wr_ref[...], preferred_element_type=jnp.float32)
        + bc_ref[...]
    )
    m = jax.nn.relu(conv)
    bi = bi_ref[...]
    bh = bh_ref[...]
    gir = jnp.dot(m, wir_ref[...], preferred_element_type=jnp.float32) + bi[0:1]
    giz = jnp.dot(m, wiz_ref[...], preferred_element_type=jnp.float32) + bi[1:2]
    gin = jnp.dot(m, win_ref[...], preferred_element_type=jnp.float32) + bi[2:3]
    ghr = jnp.dot(x, whr_ref[...], preferred_element_type=jnp.float32) + bh[0:1]
    ghz = jnp.dot(x, whz_ref[...], preferred_element_type=jnp.float32) + bh[1:2]
    ghn = jnp.dot(x, whn_ref[...], preferred_element_type=jnp.float32) + bh[2:3]
    r = jax.nn.sigmoid(gir + ghr)
    z = jax.nn.sigmoid(giz + ghz)
    n = jnp.tanh(gin + r * ghn)
    o_ref[...] = _pack4((1.0 - z) * n + z * x, BV // 4)


def _gru(aggs_a, aggs_b, xp, W_root, b_conv, wi3, wh3, bi3, bh3):
    wspec = pl.BlockSpec((DOUT, DOUT), lambda i: (0, 0))
    aspec = pl.BlockSpec((NC, BV // 4, 128), lambda i: (0, i, 0))
    return pl.pallas_call(
        _gru_body,
        grid=(VP // BV,),
        in_specs=[
            aspec,
            aspec,
            pl.BlockSpec((BV // 4, 128), lambda i: (i, 0)),
            wspec,
            pl.BlockSpec((1, DOUT), lambda i: (0, 0)),
            wspec, wspec, wspec, wspec, wspec, wspec,
            pl.BlockSpec((3, DOUT), lambda i: (0, 0)),
            pl.BlockSpec((3, DOUT), lambda i: (0, 0)),
        ],
        out_specs=pl.BlockSpec((BV // 4, 128), lambda i: (i, 0)),
        out_shape=jax.ShapeDtypeStruct((VP // 4, 128), jnp.float32),
    )(aggs_a, aggs_b, xp, W_root, b_conv.reshape(1, DOUT), wi3[0], wi3[1],
      wi3[2], wh3[0], wh3[1], wh3[2], bi3, bh3)


def _gather_half(x, src, which):
    return _gather_kernels()[which](x, src)


def _scatter_half(msg, dst3, zeros, which):
    return _scatter_kernels()[which](msg, dst3, zeros)


def kernel(node_feats, edge_attr, edge_index, W_proj, b_proj, W_e1, b_e1,
           W_e2, b_e2, W_root, b_conv, W_ih, W_hh, b_ih, b_hh):
    src = edge_index[0]
    dst_a = edge_index[1, :EA].reshape(NW, NCHA, CHUNK)
    dst_b = edge_index[1, EA:].reshape(NW, NCHB, CHUNK)
    we1 = W_e1.astype(jnp.bfloat16)
    t = jnp.arange(DOUT * DOUT)
    we2p = W_e2[:, (t % DOUT) * DOUT + t // DOUT].astype(jnp.bfloat16)
    b2m = b_e2.reshape(DOUT, DOUT).astype(jnp.bfloat16)
    li = lax.broadcasted_iota(jnp.int32, (DOUT, 128), 1)
    ri = lax.broadcasted_iota(jnp.int32, (DOUT, 128), 0)
    rep4 = (li % DOUT == ri).astype(jnp.bfloat16)
    lo = lax.broadcasted_iota(jnp.int32, (DOUT * DOUT, DOUT), 0)
    co = lax.broadcasted_iota(jnp.int32, (DOUT * DOUT, DOUT), 1)
    red = (lo // DOUT == co).astype(jnp.bfloat16)
    wi3 = W_ih.reshape(3, DOUT, DOUT).transpose(0, 2, 1)
    wh3 = W_hh.reshape(3, DOUT, DOUT).transpose(0, 2, 1)
    bi3 = b_ih.reshape(3, DOUT)
    bh3 = b_hh.reshape(3, DOUT)
    zeros = jnp.zeros((VP, DOUT), jnp.float32)

    ea_perm = edge_attr.reshape(E // TEM, TEM // 4, 4, DE).transpose(
        0, 2, 1, 3).reshape(E, DE)
    nf_pad = jnp.zeros((VP, DIN), jnp.float32).at[:V].set(node_feats)
    nf_perm = nf_pad.reshape(VP // BV, BV // 4, 4, DIN).transpose(
        0, 2, 1, 3).reshape(VP, DIN)

    wep = _wbuild(ea_perm, we1, b_e1, we2p)
    xp = _proj(nf_perm, W_proj, b_proj)
    nba = EA // TEM
    nbb = (E - EA) // TEM
    for _ in range(STEPS):
        xt = xp.reshape(VP, DOUT)
        xja = _gather_half(xt, src, 0)
        xjb = _gather_half(xt, src, 1)
        msga = _msg(wep, xja.reshape(EA // 4, 128), rep4, b2m, red, 0, nba)
        msgb = _msg(wep, xjb.reshape((E - EA) // 4, 128), rep4, b2m, red,
                    nba, nbb)
        aggs_a = _scatter_half(msga.reshape(EA, DOUT), dst_a, zeros, 0)
        aggs_b = _scatter_half(msgb.reshape(E - EA, DOUT), dst_b, zeros, 1)
        xp = _gru(aggs_a.reshape(NC, VP // 4, 128),
                  aggs_b.reshape(NC, VP // 4, 128), xp, W_root, b_conv,
                  wi3, wh3, bi3, bh3)
    return (xp.reshape(VP, DOUT)[:V], edge_attr)


# speedup vs baseline: 1.0222x; 1.0222x over previous
import functools

import jax
import jax.numpy as jnp
from jax import lax
from jax.experimental import pallas as pl
from jax.experimental.pallas import tpu as pltpu
from jax.experimental.pallas import tpu_sc as plsc

V = 10000
E = 160000
DIN = 128
DOUT = 32
DE = 16
DH = 128
STEPS = 3

VP = 10240
NC = 2
NS = 16
NW = NC * NS
CHUNK = 1000
EPW = E // NW
NCHUNK = EPW // CHUNK

TE = 4000
TEM = 4000
BV = 2048

@functools.lru_cache(maxsize=None)
def _sc_mesh():
    return plsc.VectorSubcoreMesh(
        core_axis_name="c", subcore_axis_name="s", num_cores=NC, num_subcores=NS
    )


def _pack4(y, q):
    return jnp.concatenate([y[k * q:(k + 1) * q, :] for k in range(4)], axis=1)


def _unpack4(yp):
    return jnp.concatenate(
        [yp[:, k * DOUT:(k + 1) * DOUT] for k in range(4)], axis=0)


def _proj_body(nf_ref, w_ref, b_ref, o_ref):
    y = jax.nn.relu(
        jnp.dot(nf_ref[...], w_ref[...], preferred_element_type=jnp.float32)
        + b_ref[...]
    )
    o_ref[...] = _pack4(y, BV // 4)


def _proj(nf_perm, W_proj, b_proj):
    return pl.pallas_call(
        _proj_body,
        grid=(VP // BV,),
        in_specs=[
            pl.BlockSpec((BV, DIN), lambda i: (i, 0)),
            pl.BlockSpec((DIN, DOUT), lambda i: (0, 0)),
            pl.BlockSpec((1, DOUT), lambda i: (0, 0)),
        ],
        out_specs=pl.BlockSpec((BV // 4, 128), lambda i: (i, 0)),
        out_shape=jax.ShapeDtypeStruct((VP // 4, 128), jnp.float32),
    )(nf_perm, W_proj, b_proj.reshape(1, DOUT))


def _make_gather(off, epw, nch):
    def body(x_hbm, src_hbm, out_hbm, idx_v, rows0, rows1, gs0, gs1, ws0, ws1):
        c = lax.axis_index("c")
        s = lax.axis_index("s")
        base = (c * NS + s) * epw
        pltpu.sync_copy(src_hbm.at[pl.ds(off + base, epw)], idx_v)
        bufs = (rows0, rows1)
        gsems = (gs0, gs1)
        wsems = (ws0, ws1)
        gcp = [None] * nch
        wcp = [None] * nch
        gcp[0] = pltpu.async_copy(
            x_hbm.at[idx_v.at[pl.ds(0, CHUNK)]], bufs[0], gsems[0])
        for j in range(nch):
            b = j % 2
            if j + 1 < nch:
                if j >= 1:
                    wcp[j - 1].wait()
                gcp[j + 1] = pltpu.async_copy(
                    x_hbm.at[idx_v.at[pl.ds((j + 1) * CHUNK, CHUNK)]],
                    bufs[1 - b], gsems[1 - b])
            gcp[j].wait()
            wcp[j] = pltpu.async_copy(
                bufs[b], out_hbm.at[pl.ds(base + j * CHUNK, CHUNK)], wsems[b])
        if nch >= 2:
            wcp[nch - 2].wait()
        wcp[nch - 1].wait()

    return pl.kernel(
        body,
        out_type=jax.ShapeDtypeStruct((epw * NW, DOUT), jnp.float32),
        mesh=_sc_mesh(),
        scratch_types=[
            pltpu.VMEM((epw,), jnp.int32),
            pltpu.VMEM((CHUNK, DOUT), jnp.float32),
            pltpu.VMEM((CHUNK, DOUT), jnp.float32),
            pltpu.SemaphoreType.DMA,
            pltpu.SemaphoreType.DMA,
            pltpu.SemaphoreType.DMA,
            pltpu.SemaphoreType.DMA,
        ],
        compiler_params=pltpu.CompilerParams(use_tc_tiling_on_sc=False),
    )


@functools.lru_cache(maxsize=None)
def _gather_kernel():
    return _make_gather(0, EPW, NCHUNK)


def _make_scatter(epw, nch):
    def body(msg_hbm, dst_hbm, zeros_hbm, out_hbm, idx2_v, rows0, rows1,
             acc_sh, ls0, ls1, ss0, ss1):
        c = lax.axis_index("c")
        s = lax.axis_index("s")
        rpt = VP // NS
        base = (c * NS + s) * epw
        zcp = pltpu.async_copy(
            zeros_hbm.at[pl.ds(s * rpt, rpt)], acc_sh.at[pl.ds(s * rpt, rpt)],
            ss0)
        pltpu.sync_copy(dst_hbm.at[c * NS + s], idx2_v)
        zcp.wait()
        plsc.subcore_barrier()
        bufs = (rows0, rows1)
        lsems = (ls0, ls1)
        ssems = (ss0, ss1)
        lcp = [None] * nch
        scp = [None] * nch
        lcp[0] = pltpu.async_copy(
            msg_hbm.at[pl.ds(base, CHUNK)], bufs[0], lsems[0])
        for j in range(nch):
            b = j % 2
            if j + 1 < nch:
                if j >= 1:
                    scp[j - 1].wait()
                lcp[j + 1] = pltpu.async_copy(
                    msg_hbm.at[pl.ds(base + (j + 1) * CHUNK, CHUNK)],
                    bufs[1 - b], lsems[1 - b])
            lcp[j].wait()
            scp[j] = pltpu.async_copy(
                bufs[b], acc_sh.at[idx2_v.at[j]], ssems[b], add=True)
        if nch >= 2:
            scp[nch - 2].wait()
        scp[nch - 1].wait()
        plsc.subcore_barrier()
        pltpu.sync_copy(
            acc_sh.at[pl.ds(s * rpt, rpt)], out_hbm.at[c, pl.ds(s * rpt, rpt)]
        )

    return pl.kernel(
        body,
        out_type=jax.ShapeDtypeStruct((NC, VP, DOUT), jnp.float32),
        mesh=_sc_mesh(),
        scratch_types=[
            pltpu.VMEM((nch, CHUNK), jnp.int32),
            pltpu.VMEM((CHUNK, DOUT), jnp.float32),
            pltpu.VMEM((CHUNK, DOUT), jnp.float32),
            pltpu.VMEM_SHARED((VP, DOUT), jnp.float32),
            pltpu.SemaphoreType.DMA,
            pltpu.SemaphoreType.DMA,
            pltpu.SemaphoreType.DMA,
            pltpu.SemaphoreType.DMA,
        ],
        compiler_params=pltpu.CompilerParams(use_tc_tiling_on_sc=False),
    )


@functools.lru_cache(maxsize=None)
def _scatter_kernel():
    return _make_scatter(EPW, NCHUNK)


def _wbuild_body(ea_ref, we1_ref, be1_ref, we2p_ref, o_ref):
    he = jax.nn.relu(
        jnp.dot(ea_ref[...].astype(jnp.bfloat16), we1_ref[...],
                preferred_element_type=jnp.float32)
        + be1_ref[...]
    ).astype(jnp.bfloat16)
    o_ref[...] = jnp.dot(
        he, we2p_ref[...], preferred_element_type=jnp.float32
    ).astype(jnp.bfloat16)


def _wbuild(edge_attr_f32, W_e1, b_e1, we2p):
    return pl.pallas_call(
        _wbuild_body,
        grid=(E // TE,),
        in_specs=[
            pl.BlockSpec((TE, DE), lambda i: (i, 0)),
            pl.BlockSpec((DE, DH), lambda i: (0, 0)),
            pl.BlockSpec((1, DH), lambda i: (0, 0)),
            pl.BlockSpec((DH, DOUT * DOUT), lambda i: (0, 0)),
        ],
        out_specs=pl.BlockSpec((TE, DOUT * DOUT), lambda i: (i, 0)),
        out_shape=jax.ShapeDtypeStruct((E, DOUT * DOUT), jnp.bfloat16),
    )(edge_attr_f32, W_e1, b_e1.reshape(1, DH), we2p)


def _msg_body(wep_ref, xjp_ref, rep4_ref, b2m_ref, red_ref, o_ref):
    q = TEM // 4
    we = wep_ref[...].astype(jnp.bfloat16)
    xjp = xjp_ref[...].astype(jnp.bfloat16)
    xj = jnp.concatenate(
        [xjp[:, k * DOUT:(k + 1) * DOUT] for k in range(4)], axis=0)
    xt = jnp.dot(xj, rep4_ref[...],
                 preferred_element_type=jnp.float32).astype(jnp.bfloat16)
    xr = jnp.concatenate([xt] * (DOUT * DOUT // 128), axis=1)
    p = we * xr
    msg = (
        jnp.dot(p, red_ref[...], preferred_element_type=jnp.float32)
        + jnp.dot(xj, b2m_ref[...], preferred_element_type=jnp.float32)
    )
    o_ref[...] = jnp.concatenate(
        [msg[k * q:(k + 1) * q, :] for k in range(4)], axis=1)


def _msg(wep, xjp, rep4, b2m, red, off_b, nb):
    return pl.pallas_call(
        _msg_body,
        grid=(nb,),
        in_specs=[
            pl.BlockSpec((TEM, DOUT * DOUT), lambda i: (i + off_b, 0)),
            pl.BlockSpec((TEM // 4, 128), lambda i: (i, 0)),
            pl.BlockSpec((DOUT, 128), lambda i: (0, 0)),
            pl.BlockSpec((DOUT, DOUT), lambda i: (0, 0)),
            pl.BlockSpec((DOUT * DOUT, DOUT), lambda i: (0, 0)),
        ],
        out_specs=pl.BlockSpec((TEM // 4, 128), lambda i: (i, 0)),
        out_shape=jax.ShapeDtypeStruct((nb * TEM // 4, 128), jnp.float32),
    )(wep, xjp, rep4, b2m, red)


def _gru_body(a_ref, x_ref, wr_ref, bc_ref, wir_ref, wiz_ref,
              win_ref, whr_ref, whz_ref, whn_ref, bi_ref, bh_ref, o_ref):
    x = _unpack4(x_ref[...])
    conv = (
        _unpack4(a_ref[0]) + _unpack4(a_ref[1])
        + jnp.dot(x, wr_ref[...], preferred_element_type=jnp.float32)
        + bc_ref[...]
    )
    m = jax.nn.relu(conv)
    bi = bi_ref[...]
    bh = bh_ref[...]
    gir = jnp.dot(m, wir_ref[...], preferred_element_type=jnp.float32) + bi[0:1]
    giz = jnp.dot(m, wiz_ref[...], preferred_element_type=jnp.float32) + bi[1:2]
    gin = jnp.dot(m, win_ref[...], preferred_element_type=jnp.float32) + bi[2:3]
    ghr = jnp.dot(x, whr_ref[...], preferred_element_type=jnp.float32) + bh[0:1]
    ghz = jnp.dot(x, whz_ref[...], preferred_element_type=jnp.float32) + bh[1:2]
    ghn = jnp.dot(x, whn_ref[...], preferred_element_type=jnp.float32) + bh[2:3]
    r = jax.nn.sigmoid(gir + ghr)
    z = jax.nn.sigmoid(giz + ghz)
    n = jnp.tanh(gin + r * ghn)
    o_ref[...] = _pack4((1.0 - z) * n + z * x, BV // 4)


def _gru(aggs_p, xp, W_root, b_conv, wi3, wh3, bi3, bh3):
    wspec = pl.BlockSpec((DOUT, DOUT), lambda i: (0, 0))
    aspec = pl.BlockSpec((NC, BV // 4, 128), lambda i: (0, i, 0))
    return pl.pallas_call(
        _gru_body,
        grid=(VP // BV,),
        in_specs=[
            aspec,
            pl.BlockSpec((BV // 4, 128), lambda i: (i, 0)),
            wspec,
            pl.BlockSpec((1, DOUT), lambda i: (0, 0)),
            wspec, wspec, wspec, wspec, wspec, wspec,
            pl.BlockSpec((3, DOUT), lambda i: (0, 0)),
            pl.BlockSpec((3, DOUT), lambda i: (0, 0)),
        ],
        out_specs=pl.BlockSpec((BV // 4, 128), lambda i: (i, 0)),
        out_shape=jax.ShapeDtypeStruct((VP // 4, 128), jnp.float32),
    )(aggs_p, xp, W_root, b_conv.reshape(1, DOUT), wi3[0], wi3[1],
      wi3[2], wh3[0], wh3[1], wh3[2], bi3, bh3)


def _gather(x, src):
    return _gather_kernel()(x, src)


def _scatter(msg, dst3, zeros):
    return _scatter_kernel()(msg, dst3, zeros)


def kernel(node_feats, edge_attr, edge_index, W_proj, b_proj, W_e1, b_e1,
           W_e2, b_e2, W_root, b_conv, W_ih, W_hh, b_ih, b_hh):
    src = edge_index[0]
    dst3 = edge_index[1].reshape(NW, NCHUNK, CHUNK)
    we1 = W_e1.astype(jnp.bfloat16)
    t = jnp.arange(DOUT * DOUT)
    we2p = W_e2[:, (t % DOUT) * DOUT + t // DOUT].astype(jnp.bfloat16)
    b2m = b_e2.reshape(DOUT, DOUT).astype(jnp.bfloat16)
    li = lax.broadcasted_iota(jnp.int32, (DOUT, 128), 1)
    ri = lax.broadcasted_iota(jnp.int32, (DOUT, 128), 0)
    rep4 = (li % DOUT == ri).astype(jnp.bfloat16)
    lo = lax.broadcasted_iota(jnp.int32, (DOUT * DOUT, DOUT), 0)
    co = lax.broadcasted_iota(jnp.int32, (DOUT * DOUT, DOUT), 1)
    red = (lo // DOUT == co).astype(jnp.bfloat16)
    wi3 = W_ih.reshape(3, DOUT, DOUT).transpose(0, 2, 1)
    wh3 = W_hh.reshape(3, DOUT, DOUT).transpose(0, 2, 1)
    bi3 = b_ih.reshape(3, DOUT)
    bh3 = b_hh.reshape(3, DOUT)
    zeros = jnp.zeros((VP, DOUT), jnp.float32)

    ea_perm = edge_attr.reshape(E // TEM, TEM // 4, 4, DE).transpose(
        0, 2, 1, 3).reshape(E, DE)
    nf_pad = jnp.zeros((VP, DIN), jnp.float32).at[:V].set(node_feats)
    nf_perm = nf_pad.reshape(VP // BV, BV // 4, 4, DIN).transpose(
        0, 2, 1, 3).reshape(VP, DIN)

    wep = _wbuild(ea_perm, we1, b_e1, we2p)
    xp = _proj(nf_perm, W_proj, b_proj)
    for _ in range(STEPS):
        xj = _gather(xp.reshape(VP, DOUT), src)
        msgp = _msg(wep, xj.reshape(E // 4, 128), rep4, b2m, red, 0, E // TEM)
        aggs = _scatter(msgp.reshape(E, DOUT), dst3, zeros)
        xp = _gru(aggs.reshape(NC, VP // 4, 128), xp, W_root, b_conv,
                  wi3, wh3, bi3, bh3)
    return (xp.reshape(VP, DOUT)[:V], edge_attr)
